# baseline (speedup 1.0000x reference)
import functools
import jax
import jax.numpy as jnp
from jax import lax
from jax.experimental import pallas as pl
from jax.experimental.pallas import tpu as pltpu
from jax.experimental.pallas import tpu_sc as plsc

N_NODES = 10000
N_PAD = 10240
N_EDGES = 320000
D = 64
NC = 2
NS = 16
NW = NC * NS
EDGES_PER_WORKER = N_EDGES // NW
CHUNK = 125
NCHUNKS = EDGES_PER_WORKER // CHUNK
NBUF = 4
PREF = NBUF // 2
ROWS_PER_SUB = N_PAD // NS
ZROWS = 128


def _sc_gather_scatter(tables, src3, dst3, nt):
  mesh = plsc.VectorSubcoreMesh(core_axis_name="c", subcore_axis_name="s")

  def body(tab_hbm, src_hbm, dst_hbm, out_hbm, src_v, dst_v, *rest):
    rows_vs = list(rest[:NBUF])
    zbuf = rest[NBUF]
    acc_sh = rest[NBUF + 1]
    gsems = list(rest[NBUF + 2:2 * NBUF + 2])
    ssems = list(rest[2 * NBUF + 2:])
    c = lax.axis_index("c")
    s = lax.axis_index("s")
    wid = c * NS + s
    row0 = pl.multiple_of(s * ROWS_PER_SUB, 8)

    pltpu.sync_copy(src_hbm.at[wid], src_v)
    pltpu.sync_copy(dst_hbm.at[wid], dst_v)

    zero = jnp.zeros((16,), jnp.float32)
    nsub = D // 16

    def zstore(i, carry):
      r = i // nsub
      col = (i % nsub) * 16
      zbuf[r, pl.ds(col, 16)] = zero
      return carry

    lax.fori_loop(0, ZROWS * nsub, zstore, 0)

    for t in range(nt):
      for z in range(ROWS_PER_SUB // ZROWS):
        pltpu.sync_copy(zbuf, acc_sh.at[pl.ds(row0 + z * ZROWS, ZROWS)])
      plsc.subcore_barrier()

      for b in range(PREF):
        pltpu.async_copy(tab_hbm.at[t].at[src_v.at[b]], rows_vs[b], gsems[b])

      def group_body(g, carry):
        for b in range(NBUF):
          j = g * NBUF + b
          bn = (b + PREF) % NBUF

          @pl.when(j >= PREF)
          def _():
            pltpu.make_async_copy(rows_vs[bn],
                                  acc_sh.at[dst_v.at[j - PREF]],
                                  ssems[bn]).wait()

          pltpu.make_async_copy(tab_hbm.at[t].at[src_v.at[j]], rows_vs[b],
                                gsems[b]).wait()
          pltpu.async_copy(rows_vs[b], acc_sh.at[dst_v.at[j]], ssems[b],
                           add=True)
          jn = j + PREF

          @pl.when(jn < NCHUNKS)
          def _():
            pltpu.async_copy(tab_hbm.at[t].at[src_v.at[jn]], rows_vs[bn],
                             gsems[bn])
        return carry

      lax.fori_loop(0, NCHUNKS // NBUF, group_body, 0)
      for i in range(PREF):
        j = NCHUNKS - PREF + i
        pltpu.make_async_copy(rows_vs[j % NBUF], acc_sh.at[dst_v.at[j]],
                              ssems[j % NBUF]).wait()
      plsc.subcore_barrier()

      pltpu.sync_copy(acc_sh.at[pl.ds(row0, ROWS_PER_SUB)],
                      out_hbm.at[t, c, pl.ds(row0, ROWS_PER_SUB)])
      plsc.subcore_barrier()

  k = pl.kernel(
      body,
      out_type=jax.ShapeDtypeStruct((nt, NC, N_PAD, D), jnp.float32),
      mesh=mesh,
      compiler_params=pltpu.CompilerParams(use_tc_tiling_on_sc=False),
      scratch_types=[
          pltpu.VMEM((NCHUNKS, CHUNK), jnp.int32),
          pltpu.VMEM((NCHUNKS, CHUNK), jnp.int32),
          *([pltpu.VMEM((CHUNK, D), jnp.float32)] * NBUF),
          pltpu.VMEM((ZROWS, D), jnp.float32),
          pltpu.VMEM_SHARED((N_PAD, D), jnp.float32),
          *([pltpu.SemaphoreType.DMA] * (2 * NBUF)),
      ],
  )
  return k(tables, src3, dst3)


def _tc_matmul_split(x, w):
  bm = 1000
  n, kin = x.shape
  ws = jnp.stack([w[:, :D], w[:, D:]])

  def body(x_ref, w_ref, o_ref):
    o_ref[0] = jnp.dot(x_ref[...], w_ref[0],
                       preferred_element_type=jnp.float32)

  return pl.pallas_call(
      body,
      grid=(2, n // bm),
      in_specs=[
          pl.BlockSpec((bm, kin), lambda j, i: (i, 0)),
          pl.BlockSpec((1, kin, D), lambda j, i: (j, 0, 0)),
      ],
      out_specs=pl.BlockSpec((1, bm, D), lambda j, i: (j, i, 0)),
      out_shape=jax.ShapeDtypeStruct((2, n, D), jnp.float32),
  )(x, ws)


def _tc_fuse_relu_matmul(parts, b, w):
  bm = 1000
  kout = w.shape[1]

  def body(p_ref, b_ref, w_ref, o_ref):
    p = p_ref[...]
    h = jnp.concatenate([p[0, 0] + p[0, 1], p[1, 0] + p[1, 1]], axis=-1)
    h = jax.nn.relu(h + b_ref[...])
    o_ref[...] = jnp.dot(h, w_ref[...], preferred_element_type=jnp.float32)

  return pl.pallas_call(
      body,
      grid=(N_NODES // bm,),
      in_specs=[
          pl.BlockSpec((2, NC, bm, D), lambda i: (0, 0, i, 0)),
          pl.BlockSpec((1, 2 * D), lambda i: (0, 0)),
          pl.BlockSpec((2 * D, kout), lambda i: (0, 0)),
      ],
      out_specs=pl.BlockSpec((bm, kout), lambda i: (i, 0)),
      out_shape=jax.ShapeDtypeStruct((N_NODES, kout), jnp.float32),
  )(parts, b.reshape(1, 2 * D), w)


def _tc_sum_bias(parts, b):
  bm = 1000

  def body(p_ref, b_ref, o_ref):
    p = p_ref[...]
    o_ref[...] = p[0, 0] + p[0, 1] + b_ref[...]

  return pl.pallas_call(
      body,
      grid=(N_NODES // bm,),
      in_specs=[
          pl.BlockSpec((1, NC, bm, D), lambda i: (0, 0, i, 0)),
          pl.BlockSpec((1, D), lambda i: (0, 0)),
      ],
      out_specs=pl.BlockSpec((bm, D), lambda i: (i, 0)),
      out_shape=jax.ShapeDtypeStruct((N_NODES, D), jnp.float32),
  )(parts, b.reshape(1, D))


def kernel(x, adj, W1, b1, W2, b2):
  src3 = adj[0].reshape(NW, NCHUNKS, CHUNK)
  dst3 = adj[1].reshape(NW, NCHUNKS, CHUNK)

  h = _tc_matmul_split(x, W1)
  parts1 = _sc_gather_scatter(h, src3, dst3, 2)
  g = _tc_fuse_relu_matmul(parts1, b1, W2)
  parts2 = _sc_gather_scatter(g[None], src3, dst3, 1)
  return _tc_sum_bias(parts2, b2)


# speedup vs baseline: 1.1333x; 1.1333x over previous
import functools
import jax
import jax.numpy as jnp
from jax import lax
from jax.experimental import pallas as pl
from jax.experimental.pallas import tpu as pltpu
from jax.experimental.pallas import tpu_sc as plsc

N_NODES = 10000
N_PAD = 10240
N_EDGES = 320000
D = 64
NC = 2
NS = 16
NW = NC * NS
EDGES_PER_WORKER = N_EDGES // NW
CHUNK = 125
NCHUNKS = EDGES_PER_WORKER // CHUNK
NBUF = 4
PREF = 3
DRAIN = 1
ROWS_PER_SUB = N_PAD // NS
ZROWS = 128


def _sc_gather_scatter(tables, src3, dst3, nt):
  mesh = plsc.VectorSubcoreMesh(core_axis_name="c", subcore_axis_name="s")

  def body(tab_hbm, src_hbm, dst_hbm, out_hbm, src_v, dst_v, *rest):
    rows_vs = list(rest[:NBUF])
    zbuf = rest[NBUF]
    acc_sh = rest[NBUF + 1]
    gsems = list(rest[NBUF + 2:2 * NBUF + 2])
    ssems = list(rest[2 * NBUF + 2:])
    c = lax.axis_index("c")
    s = lax.axis_index("s")
    wid = c * NS + s
    row0 = pl.multiple_of(s * ROWS_PER_SUB, 8)

    pltpu.sync_copy(src_hbm.at[wid], src_v)
    pltpu.sync_copy(dst_hbm.at[wid], dst_v)

    zero = jnp.zeros((16,), jnp.float32)
    nsub = D // 16

    def zstore(i, carry):
      r = i // nsub
      col = (i % nsub) * 16
      zbuf[r, pl.ds(col, 16)] = zero
      return carry

    lax.fori_loop(0, ZROWS * nsub, zstore, 0)

    for t in range(nt):
      for z in range(ROWS_PER_SUB // ZROWS):
        pltpu.sync_copy(zbuf, acc_sh.at[pl.ds(row0 + z * ZROWS, ZROWS)])
      plsc.subcore_barrier()

      for b in range(PREF):
        pltpu.async_copy(tab_hbm.at[t].at[src_v.at[b]], rows_vs[b], gsems[b])

      def group_body(g, carry):
        for b in range(NBUF):
          j = g * NBUF + b
          pltpu.make_async_copy(tab_hbm.at[t].at[src_v.at[j]], rows_vs[b],
                                gsems[b]).wait()
          bd = (b - DRAIN) % NBUF

          @pl.when(j >= DRAIN)
          def _():
            pltpu.make_async_copy(rows_vs[bd],
                                  acc_sh.at[dst_v.at[j - DRAIN]],
                                  ssems[bd]).wait()

          pltpu.async_copy(rows_vs[b], acc_sh.at[dst_v.at[j]], ssems[b],
                           add=True)
          jn = j + PREF
          bg = (b + PREF) % NBUF

          @pl.when(jn < NCHUNKS)
          def _():
            pltpu.async_copy(tab_hbm.at[t].at[src_v.at[jn]], rows_vs[bg],
                             gsems[bg])
        return carry

      lax.fori_loop(0, NCHUNKS // NBUF, group_body, 0)
      for i in range(DRAIN):
        j = NCHUNKS - DRAIN + i
        pltpu.make_async_copy(rows_vs[j % NBUF], acc_sh.at[dst_v.at[j]],
                              ssems[j % NBUF]).wait()
      plsc.subcore_barrier()

      pltpu.sync_copy(acc_sh.at[pl.ds(row0, ROWS_PER_SUB)],
                      out_hbm.at[t, c, pl.ds(row0, ROWS_PER_SUB)])
      plsc.subcore_barrier()

  k = pl.kernel(
      body,
      out_type=jax.ShapeDtypeStruct((nt, NC, N_PAD, D), jnp.float32),
      mesh=mesh,
      compiler_params=pltpu.CompilerParams(use_tc_tiling_on_sc=False),
      scratch_types=[
          pltpu.VMEM((NCHUNKS, CHUNK), jnp.int32),
          pltpu.VMEM((NCHUNKS, CHUNK), jnp.int32),
          *([pltpu.VMEM((CHUNK, D), jnp.float32)] * NBUF),
          pltpu.VMEM((ZROWS, D), jnp.float32),
          pltpu.VMEM_SHARED((N_PAD, D), jnp.float32),
          *([pltpu.SemaphoreType.DMA] * (2 * NBUF)),
      ],
  )
  return k(tables, src3, dst3)


def _tc_matmul_split(x, w):
  bm = 1000
  n, kin = x.shape
  ws = jnp.stack([w[:, :D], w[:, D:]])

  def body(x_ref, w_ref, o_ref):
    o_ref[0] = jnp.dot(x_ref[...], w_ref[0],
                       preferred_element_type=jnp.float32)

  return pl.pallas_call(
      body,
      grid=(2, n // bm),
      in_specs=[
          pl.BlockSpec((bm, kin), lambda j, i: (i, 0)),
          pl.BlockSpec((1, kin, D), lambda j, i: (j, 0, 0)),
      ],
      out_specs=pl.BlockSpec((1, bm, D), lambda j, i: (j, i, 0)),
      out_shape=jax.ShapeDtypeStruct((2, n, D), jnp.float32),
  )(x, ws)


def _tc_fuse_relu_matmul(parts, b, w):
  bm = 1000
  kout = w.shape[1]

  def body(p_ref, b_ref, w_ref, o_ref):
    p = p_ref[...]
    h = jnp.concatenate([p[0, 0] + p[0, 1], p[1, 0] + p[1, 1]], axis=-1)
    h = jax.nn.relu(h + b_ref[...])
    o_ref[...] = jnp.dot(h, w_ref[...], preferred_element_type=jnp.float32)

  return pl.pallas_call(
      body,
      grid=(N_NODES // bm,),
      in_specs=[
          pl.BlockSpec((2, NC, bm, D), lambda i: (0, 0, i, 0)),
          pl.BlockSpec((1, 2 * D), lambda i: (0, 0)),
          pl.BlockSpec((2 * D, kout), lambda i: (0, 0)),
      ],
      out_specs=pl.BlockSpec((bm, kout), lambda i: (i, 0)),
      out_shape=jax.ShapeDtypeStruct((N_NODES, kout), jnp.float32),
  )(parts, b.reshape(1, 2 * D), w)


def _tc_sum_bias(parts, b):
  bm = 1000

  def body(p_ref, b_ref, o_ref):
    p = p_ref[...]
    o_ref[...] = p[0, 0] + p[0, 1] + b_ref[...]

  return pl.pallas_call(
      body,
      grid=(N_NODES // bm,),
      in_specs=[
          pl.BlockSpec((1, NC, bm, D), lambda i: (0, 0, i, 0)),
          pl.BlockSpec((1, D), lambda i: (0, 0)),
      ],
      out_specs=pl.BlockSpec((bm, D), lambda i: (i, 0)),
      out_shape=jax.ShapeDtypeStruct((N_NODES, D), jnp.float32),
  )(parts, b.reshape(1, D))


def kernel(x, adj, W1, b1, W2, b2):
  src3 = adj[0].reshape(NW, NCHUNKS, CHUNK)
  dst3 = adj[1].reshape(NW, NCHUNKS, CHUNK)

  h = _tc_matmul_split(x, W1)
  parts1 = _sc_gather_scatter(h, src3, dst3, 2)
  g = _tc_fuse_relu_matmul(parts1, b1, W2)
  parts2 = _sc_gather_scatter(g[None], src3, dst3, 1)
  return _tc_sum_bias(parts2, b2)


# speedup vs baseline: 1.4858x; 1.3110x over previous
import functools
import jax
import jax.numpy as jnp
from jax import lax
from jax.experimental import pallas as pl
from jax.experimental.pallas import tpu as pltpu
from jax.experimental.pallas import tpu_sc as plsc

N_NODES = 10000
N_PAD = 10240
N_EDGES = 320000
D = 64
NC = 2
NS = 16
NW = NC * NS
EDGES_PER_WORKER = N_EDGES // NW
CHUNK = 80
NCHUNKS = EDGES_PER_WORKER // CHUNK
NBUF = 5
PREF = 4
DRAIN = 1
ROWS_PER_SUB = N_PAD // NS
ZROWS = 128
VPR = CHUNK // 16


def _sc_gather_scatter(table, src3, dst3, nt, nrows, double_idx):
  mesh = plsc.VectorSubcoreMesh(core_axis_name="c", subcore_axis_name="s")

  def body(tab_hbm, src_hbm, dst_hbm, out_hbm, src_v, dst_v, *rest):
    rows_vs = list(rest[:NBUF])
    zbuf = rest[NBUF]
    acc_sh = rest[NBUF + 1]
    gsems = list(rest[NBUF + 2:2 * NBUF + 2])
    ssems = list(rest[2 * NBUF + 2:])
    c = lax.axis_index("c")
    s = lax.axis_index("s")
    wid = c * NS + s
    row0 = pl.multiple_of(s * ROWS_PER_SUB, 8)

    pltpu.sync_copy(src_hbm.at[wid], src_v)
    pltpu.sync_copy(dst_hbm.at[wid], dst_v)

    zero = jnp.zeros((16,), jnp.float32)
    nsub = D // 16

    def zstore(i, carry):
      r = i // nsub
      col = (i % nsub) * 16
      zbuf[r, pl.ds(col, 16)] = zero
      return carry

    lax.fori_loop(0, ZROWS * nsub, zstore, 0)

    for t in range(nt):
      if double_idx:
        def itrans(r, carry):
          for k in range(VPR):
            v = src_v[r, pl.ds(k * 16, 16)]
            src_v[r, pl.ds(k * 16, 16)] = (v + v) if t == 0 else (v + 1)
          return carry

        lax.fori_loop(0, NCHUNKS, itrans, 0)

      for z in range(ROWS_PER_SUB // ZROWS):
        pltpu.sync_copy(zbuf, acc_sh.at[pl.ds(row0 + z * ZROWS, ZROWS)])
      plsc.subcore_barrier()

      for b in range(PREF):
        pltpu.async_copy(tab_hbm.at[src_v.at[b]], rows_vs[b], gsems[b])

      def group_body(g, carry):
        for b in range(NBUF):
          j = g * NBUF + b
          pltpu.make_async_copy(tab_hbm.at[src_v.at[j]], rows_vs[b],
                                gsems[b]).wait()
          bd = (b - DRAIN) % NBUF

          @pl.when(j >= DRAIN)
          def _():
            pltpu.make_async_copy(rows_vs[bd],
                                  acc_sh.at[dst_v.at[j - DRAIN]],
                                  ssems[bd]).wait()

          pltpu.async_copy(rows_vs[b], acc_sh.at[dst_v.at[j]], ssems[b],
                           add=True)
          jn = j + PREF
          bg = (b + PREF) % NBUF

          @pl.when(jn < NCHUNKS)
          def _():
            pltpu.async_copy(tab_hbm.at[src_v.at[jn]], rows_vs[bg],
                             gsems[bg])
        return carry

      lax.fori_loop(0, NCHUNKS // NBUF, group_body, 0)
      for i in range(DRAIN):
        j = NCHUNKS - DRAIN + i
        pltpu.make_async_copy(rows_vs[j % NBUF], acc_sh.at[dst_v.at[j]],
                              ssems[j % NBUF]).wait()
      plsc.subcore_barrier()

      pltpu.sync_copy(acc_sh.at[pl.ds(row0, ROWS_PER_SUB)],
                      out_hbm.at[t, c, pl.ds(row0, ROWS_PER_SUB)])
      plsc.subcore_barrier()

  k = pl.kernel(
      body,
      out_type=jax.ShapeDtypeStruct((nt, NC, N_PAD, D), jnp.float32),
      mesh=mesh,
      compiler_params=pltpu.CompilerParams(use_tc_tiling_on_sc=False),
      scratch_types=[
          pltpu.VMEM((NCHUNKS, CHUNK), jnp.int32),
          pltpu.VMEM((NCHUNKS, CHUNK), jnp.int32),
          *([pltpu.VMEM((CHUNK, D), jnp.float32)] * NBUF),
          pltpu.VMEM((ZROWS, D), jnp.float32),
          pltpu.VMEM_SHARED((N_PAD, D), jnp.float32),
          *([pltpu.SemaphoreType.DMA] * (2 * NBUF)),
      ],
  )
  return k(table, src3, dst3)


def _tc_matmul(x, w):
  bm = 1000
  n, kin = x.shape
  kout = w.shape[1]

  def body(x_ref, w_ref, o_ref):
    o_ref[...] = jnp.dot(x_ref[...], w_ref[...],
                         preferred_element_type=jnp.float32)

  return pl.pallas_call(
      body,
      grid=(n // bm,),
      in_specs=[
          pl.BlockSpec((bm, kin), lambda i: (i, 0)),
          pl.BlockSpec((kin, kout), lambda i: (0, 0)),
      ],
      out_specs=pl.BlockSpec((bm, kout), lambda i: (i, 0)),
      out_shape=jax.ShapeDtypeStruct((n, kout), jnp.float32),
  )(x, w)


def _tc_fuse_relu_matmul(parts128, ba, bb, wa, wb):
  bm = 1000
  nrows = N_NODES // 2

  def body(p_ref, ba_ref, bb_ref, wa_ref, wb_ref, o_ref):
    p = p_ref[...]
    a = jax.nn.relu(p[0, 0] + p[0, 1] + ba_ref[...])
    b = jax.nn.relu(p[1, 0] + p[1, 1] + bb_ref[...])
    o_ref[...] = (
        jnp.dot(a, wa_ref[...], preferred_element_type=jnp.float32)
        + jnp.dot(b, wb_ref[...], preferred_element_type=jnp.float32))

  return pl.pallas_call(
      body,
      grid=(nrows // bm,),
      in_specs=[
          pl.BlockSpec((2, NC, bm, 128), lambda i: (0, 0, i, 0)),
          pl.BlockSpec((1, 128), lambda i: (0, 0)),
          pl.BlockSpec((1, 128), lambda i: (0, 0)),
          pl.BlockSpec((128, 128), lambda i: (0, 0)),
          pl.BlockSpec((128, 128), lambda i: (0, 0)),
      ],
      out_specs=pl.BlockSpec((bm, 128), lambda i: (i, 0)),
      out_shape=jax.ShapeDtypeStruct((nrows, 128), jnp.float32),
  )(parts128, ba, bb, wa, wb)


def _tc_sum_bias(parts128, b2t):
  bm = 1000
  nrows = N_NODES // 2

  def body(p_ref, b_ref, o_ref):
    p = p_ref[...]
    o_ref[...] = p[0, 0] + p[0, 1] + b_ref[...]

  return pl.pallas_call(
      body,
      grid=(nrows // bm,),
      in_specs=[
          pl.BlockSpec((1, NC, bm, 128), lambda i: (0, 0, i, 0)),
          pl.BlockSpec((1, 128), lambda i: (0, 0)),
      ],
      out_specs=pl.BlockSpec((bm, 128), lambda i: (i, 0)),
      out_shape=jax.ShapeDtypeStruct((nrows, 128), jnp.float32),
  )(parts128, b2t)


def kernel(x, adj, W1, b1, W2, b2):
  src3 = adj[0].reshape(NW, NCHUNKS, CHUNK)
  dst3 = adj[1].reshape(NW, NCHUNKS, CHUNK)

  zero64 = jnp.zeros((64, 64), jnp.float32)
  wa = jnp.block([[W2[:64], zero64], [zero64, W2[:64]]])
  wb = jnp.block([[W2[64:], zero64], [zero64, W2[64:]]])
  ba = jnp.concatenate([b1[:64], b1[:64]]).reshape(1, 128)
  bb = jnp.concatenate([b1[64:], b1[64:]]).reshape(1, 128)
  b2t = jnp.concatenate([b2, b2]).reshape(1, 128)

  h = _tc_matmul(x, W1)
  h64 = h.reshape(2 * N_NODES, D)
  parts1 = _sc_gather_scatter(h64, src3, dst3, 2, 2 * N_NODES, True)
  p1v = parts1.reshape(2, NC, N_PAD // 2, 128)
  g128 = _tc_fuse_relu_matmul(p1v, ba, bb, wa, wb)
  g64 = g128.reshape(N_NODES, D)
  parts2 = _sc_gather_scatter(g64, src3, dst3, 1, N_NODES, False)
  p2v = parts2.reshape(1, NC, N_PAD // 2, 128)
  out128 = _tc_sum_bias(p2v, b2t)
  return out128.reshape(N_NODES, 64)


# speedup vs baseline: 1.6047x; 1.0800x over previous
import functools
import jax
import jax.numpy as jnp
from jax import lax
from jax.experimental import pallas as pl
from jax.experimental.pallas import tpu as pltpu
from jax.experimental.pallas import tpu_sc as plsc

N_NODES = 10000
N_PAD = 10240
N_EDGES = 320000
D = 64
NC = 2
NS = 16
NW = NC * NS
CHUNK = 128
NBLK = N_EDGES // CHUNK
NITER = 84
NBUF = 6
PREF = 4
IPREF = 6
ROWS_PER_SUB = N_PAD // NS
ZROWS = 128


def _sc_gather_scatter(table, adjv, nt, double_idx):
  mesh = plsc.VectorSubcoreMesh(core_axis_name="c", subcore_axis_name="s")

  def body(tab_hbm, adj_hbm, out_hbm, *rest):
    idx_vs = list(rest[:NBUF])
    rows_vs = list(rest[NBUF:2 * NBUF])
    zbuf = rest[2 * NBUF]
    acc_sh = rest[2 * NBUF + 1]
    isems = list(rest[2 * NBUF + 2:3 * NBUF + 2])
    gsems = list(rest[3 * NBUF + 2:])
    c = lax.axis_index("c")
    s = lax.axis_index("s")
    wid = c * NS + s
    row0 = pl.multiple_of(s * ROWS_PER_SUB, 8)
    nw = jnp.where(wid < NBLK - (NBLK // NW) * NW, NBLK // NW + 1, NBLK // NW)

    def blk(i):
      return wid + i * NW

    zero = jnp.zeros((16,), jnp.float32)
    nsub = D // 16

    def zstore(i, carry):
      r = i // nsub
      col = (i % nsub) * 16
      zbuf[r, pl.ds(col, 16)] = zero
      return carry

    lax.fori_loop(0, ZROWS * nsub, zstore, 0)

    def stage_idx(j, b):
      pltpu.async_copy(adj_hbm.at[blk(j)], idx_vs[b], isems[b])

    def gather(j, b, t):
      pltpu.make_async_copy(adj_hbm.at[blk(j)], idx_vs[b], isems[b]).wait()
      if double_idx:
        for k in range(CHUNK // 16):
          v = idx_vs[b][0, pl.ds(k * 16, 16)]
          v = v + v
          if t:
            v = v + 1
          idx_vs[b][0, pl.ds(k * 16, 16)] = v
      pltpu.async_copy(tab_hbm.at[idx_vs[b].at[0]], rows_vs[b], gsems[b])

    for t in range(nt):
      for z in range(ROWS_PER_SUB // ZROWS):
        pltpu.sync_copy(zbuf, acc_sh.at[pl.ds(row0 + z * ZROWS, ZROWS)])
      plsc.subcore_barrier()

      for j in range(IPREF):
        stage_idx(j, j % NBUF)
      for j in range(PREF):
        gather(j, j % NBUF, t)

      def group_body(g, carry):
        for b in range(NBUF):
          j = g * NBUF + b

          @pl.when(j < nw)
          def _():
            pltpu.make_async_copy(tab_hbm.at[idx_vs[b].at[0]], rows_vs[b],
                                  gsems[b]).wait()
            pltpu.sync_copy(rows_vs[b], acc_sh.at[idx_vs[b].at[1]], add=True)

          bg = (b + PREF) % NBUF

          @pl.when(j + PREF < nw)
          def _():
            gather(j + PREF, bg, t)

          @pl.when(j + IPREF < nw)
          def _():
            stage_idx(j + IPREF, b)
        return carry

      lax.fori_loop(0, NITER // NBUF, group_body, 0)
      plsc.subcore_barrier()

      pltpu.sync_copy(acc_sh.at[pl.ds(row0, ROWS_PER_SUB)],
                      out_hbm.at[t, c, pl.ds(row0, ROWS_PER_SUB)])
      plsc.subcore_barrier()

  k = pl.kernel(
      body,
      out_type=jax.ShapeDtypeStruct((nt, NC, N_PAD, D), jnp.float32),
      mesh=mesh,
      compiler_params=pltpu.CompilerParams(use_tc_tiling_on_sc=False),
      scratch_types=[
          *([pltpu.VMEM((2, CHUNK), jnp.int32)] * NBUF),
          *([pltpu.VMEM((CHUNK, D), jnp.float32)] * NBUF),
          pltpu.VMEM((ZROWS, D), jnp.float32),
          pltpu.VMEM_SHARED((N_PAD, D), jnp.float32),
          *([pltpu.SemaphoreType.DMA] * (2 * NBUF)),
      ],
  )
  return k(table, adjv)


def _tc_matmul(x, w):
  bm = 1000
  n, kin = x.shape
  kout = w.shape[1]

  def body(x_ref, w_ref, o_ref):
    o_ref[...] = jnp.dot(x_ref[...], w_ref[...],
                         preferred_element_type=jnp.float32)

  return pl.pallas_call(
      body,
      grid=(n // bm,),
      in_specs=[
          pl.BlockSpec((bm, kin), lambda i: (i, 0)),
          pl.BlockSpec((kin, kout), lambda i: (0, 0)),
      ],
      out_specs=pl.BlockSpec((bm, kout), lambda i: (i, 0)),
      out_shape=jax.ShapeDtypeStruct((n, kout), jnp.float32),
  )(x, w)


def _tc_fuse_relu_matmul(parts128, ba, bb, wa, wb):
  bm = 1000
  nrows = N_NODES // 2

  def body(p_ref, ba_ref, bb_ref, wa_ref, wb_ref, o_ref):
    p = p_ref[...]
    a = jax.nn.relu(p[0, 0] + p[0, 1] + ba_ref[...])
    b = jax.nn.relu(p[1, 0] + p[1, 1] + bb_ref[...])
    o_ref[...] = (
        jnp.dot(a, wa_ref[...], preferred_element_type=jnp.float32)
        + jnp.dot(b, wb_ref[...], preferred_element_type=jnp.float32))

  return pl.pallas_call(
      body,
      grid=(nrows // bm,),
      in_specs=[
          pl.BlockSpec((2, NC, bm, 128), lambda i: (0, 0, i, 0)),
          pl.BlockSpec((1, 128), lambda i: (0, 0)),
          pl.BlockSpec((1, 128), lambda i: (0, 0)),
          pl.BlockSpec((128, 128), lambda i: (0, 0)),
          pl.BlockSpec((128, 128), lambda i: (0, 0)),
      ],
      out_specs=pl.BlockSpec((bm, 128), lambda i: (i, 0)),
      out_shape=jax.ShapeDtypeStruct((nrows, 128), jnp.float32),
  )(parts128, ba, bb, wa, wb)


def _tc_sum_bias(parts128, b2t):
  bm = 1000
  nrows = N_NODES // 2

  def body(p_ref, b_ref, o_ref):
    p = p_ref[...]
    o_ref[...] = p[0, 0] + p[0, 1] + b_ref[...]

  return pl.pallas_call(
      body,
      grid=(nrows // bm,),
      in_specs=[
          pl.BlockSpec((1, NC, bm, 128), lambda i: (0, 0, i, 0)),
          pl.BlockSpec((1, 128), lambda i: (0, 0)),
      ],
      out_specs=pl.BlockSpec((bm, 128), lambda i: (i, 0)),
      out_shape=jax.ShapeDtypeStruct((nrows, 128), jnp.float32),
  )(parts128, b2t)


def kernel(x, adj, W1, b1, W2, b2):
  adjv = adj.reshape(2, NBLK, CHUNK).transpose(1, 0, 2)

  zero64 = jnp.zeros((64, 64), jnp.float32)
  wa = jnp.block([[W2[:64], zero64], [zero64, W2[:64]]])
  wb = jnp.block([[W2[64:], zero64], [zero64, W2[64:]]])
  ba = jnp.concatenate([b1[:64], b1[:64]]).reshape(1, 128)
  bb = jnp.concatenate([b1[64:], b1[64:]]).reshape(1, 128)
  b2t = jnp.concatenate([b2, b2]).reshape(1, 128)

  h = _tc_matmul(x, W1)
  h64 = h.reshape(2 * N_NODES, D)
  parts1 = _sc_gather_scatter(h64, adjv, 2, True)
  p1v = parts1.reshape(2, NC, N_PAD // 2, 128)
  g128 = _tc_fuse_relu_matmul(p1v, ba, bb, wa, wb)
  g64 = g128.reshape(N_NODES, D)
  parts2 = _sc_gather_scatter(g64, adjv, 1, False)
  p2v = parts2.reshape(1, NC, N_PAD // 2, 128)
  out128 = _tc_sum_bias(p2v, b2t)
  return out128.reshape(N_NODES, 64)


# speedup vs baseline: 1.6275x; 1.0142x over previous
import functools
import jax
import jax.numpy as jnp
from jax import lax
from jax.experimental import pallas as pl
from jax.experimental.pallas import tpu as pltpu
from jax.experimental.pallas import tpu_sc as plsc

N_NODES = 10000
N_PAD = 10240
N_EDGES = 320000
D = 64
NC = 2
NS = 16
NW = NC * NS
CHUNK = 128
NBLK = N_EDGES // CHUNK
NITER = 84
NBUF = 6
PREF = 4
IPREF = 6
ROWS_PER_SUB = N_PAD // NS
ZROWS = 128


def _sc_gather_scatter(table, adjv, nt, double_idx):
  mesh = plsc.VectorSubcoreMesh(core_axis_name="c", subcore_axis_name="s")

  def body(tab_hbm, adj_hbm, out_hbm, *rest):
    idx_vs = list(rest[:NBUF])
    rows_vs = list(rest[NBUF:2 * NBUF])
    zbuf = rest[2 * NBUF]
    acc_sh = rest[2 * NBUF + 1]
    isems = list(rest[2 * NBUF + 2:3 * NBUF + 2])
    gsems = list(rest[3 * NBUF + 2:])
    c = lax.axis_index("c")
    s = lax.axis_index("s")
    wid = c * NS + s
    row0 = pl.multiple_of(s * ROWS_PER_SUB, 8)
    nw = jnp.where(wid < NBLK - (NBLK // NW) * NW, NBLK // NW + 1, NBLK // NW)

    def blk(i):
      return wid + i * NW

    zero = jnp.zeros((16,), jnp.float32)
    nsub = D // 16

    def zstore(i, carry):
      r = i // nsub
      col = (i % nsub) * 16
      zbuf[r, pl.ds(col, 16)] = zero
      return carry

    lax.fori_loop(0, ZROWS * nsub, zstore, 0)

    def stage_idx(j, b):
      pltpu.async_copy(adj_hbm.at[blk(j)], idx_vs[b], isems[b])

    def gather(j, b, t):
      pltpu.make_async_copy(adj_hbm.at[blk(j)], idx_vs[b], isems[b]).wait()
      if double_idx:
        for k in range(CHUNK // 16):
          v = idx_vs[b][0, pl.ds(k * 16, 16)]
          v = v + v
          if t:
            v = v + 1
          idx_vs[b][0, pl.ds(k * 16, 16)] = v
      pltpu.async_copy(tab_hbm.at[idx_vs[b].at[0]], rows_vs[b], gsems[b])

    for t in range(nt):
      for z in range(ROWS_PER_SUB // ZROWS):
        pltpu.sync_copy(zbuf, acc_sh.at[pl.ds(row0 + z * ZROWS, ZROWS)])
      plsc.subcore_barrier()

      for j in range(IPREF):
        stage_idx(j, j % NBUF)
      for j in range(PREF):
        gather(j, j % NBUF, t)

      def group_body(g, carry):
        for b in range(NBUF):
          j = g * NBUF + b

          @pl.when(j < nw)
          def _():
            pltpu.make_async_copy(tab_hbm.at[idx_vs[b].at[0]], rows_vs[b],
                                  gsems[b]).wait()
            pltpu.sync_copy(rows_vs[b], acc_sh.at[idx_vs[b].at[1]], add=True)

          bg = (b + PREF) % NBUF

          @pl.when(j + PREF < nw)
          def _():
            gather(j + PREF, bg, t)

          @pl.when(j + IPREF < nw)
          def _():
            stage_idx(j + IPREF, b)
        return carry

      lax.fori_loop(0, NITER // NBUF, group_body, 0)
      plsc.subcore_barrier()

      pltpu.sync_copy(acc_sh.at[pl.ds(row0, ROWS_PER_SUB)],
                      out_hbm.at[t, c, pl.ds(row0, ROWS_PER_SUB)])
      plsc.subcore_barrier()

  k = pl.kernel(
      body,
      out_type=jax.ShapeDtypeStruct((nt, NC, N_PAD, D), jnp.float32),
      mesh=mesh,
      compiler_params=pltpu.CompilerParams(use_tc_tiling_on_sc=False),
      scratch_types=[
          *([pltpu.VMEM((2, CHUNK), jnp.int32)] * NBUF),
          *([pltpu.VMEM((CHUNK, D), jnp.float32)] * NBUF),
          pltpu.VMEM((ZROWS, D), jnp.float32),
          pltpu.VMEM_SHARED((N_PAD, D), jnp.float32),
          *([pltpu.SemaphoreType.DMA] * (2 * NBUF)),
      ],
  )
  return k(table, adjv)


def _tc_matmul(x, w):
  bm = 2000
  n, kin = x.shape
  kout = w.shape[1]

  def body(x_ref, w_ref, o_ref):
    o_ref[...] = jnp.dot(x_ref[...], w_ref[...],
                         preferred_element_type=jnp.float32)

  return pl.pallas_call(
      body,
      grid=(n // bm,),
      in_specs=[
          pl.BlockSpec((bm, kin), lambda i: (i, 0)),
          pl.BlockSpec((kin, kout), lambda i: (0, 0)),
      ],
      out_specs=pl.BlockSpec((bm, kout), lambda i: (i, 0)),
      out_shape=jax.ShapeDtypeStruct((n, kout), jnp.float32),
  )(x, w)


def _tc_fuse_relu_matmul(parts128, ba, bb, wa, wb):
  bm = 1000
  nrows = N_NODES // 2

  def body(p_ref, ba_ref, bb_ref, wa_ref, wb_ref, o_ref):
    p = p_ref[...]
    a = jax.nn.relu(p[0, 0] + p[0, 1] + ba_ref[...])
    b = jax.nn.relu(p[1, 0] + p[1, 1] + bb_ref[...])
    o_ref[...] = (
        jnp.dot(a, wa_ref[...], preferred_element_type=jnp.float32)
        + jnp.dot(b, wb_ref[...], preferred_element_type=jnp.float32))

  return pl.pallas_call(
      body,
      grid=(nrows // bm,),
      in_specs=[
          pl.BlockSpec((2, NC, bm, 128), lambda i: (0, 0, i, 0)),
          pl.BlockSpec((1, 128), lambda i: (0, 0)),
          pl.BlockSpec((1, 128), lambda i: (0, 0)),
          pl.BlockSpec((128, 128), lambda i: (0, 0)),
          pl.BlockSpec((128, 128), lambda i: (0, 0)),
      ],
      out_specs=pl.BlockSpec((bm, 128), lambda i: (i, 0)),
      out_shape=jax.ShapeDtypeStruct((nrows, 128), jnp.float32),
  )(parts128, ba, bb, wa, wb)


def _tc_sum_bias(parts128, b2t):
  bm = 1000
  nrows = N_NODES // 2

  def body(p_ref, b_ref, o_ref):
    p = p_ref[...]
    o_ref[...] = p[0, 0] + p[0, 1] + b_ref[...]

  return pl.pallas_call(
      body,
      grid=(nrows // bm,),
      in_specs=[
          pl.BlockSpec((1, NC, bm, 128), lambda i: (0, 0, i, 0)),
          pl.BlockSpec((1, 128), lambda i: (0, 0)),
      ],
      out_specs=pl.BlockSpec((bm, 128), lambda i: (i, 0)),
      out_shape=jax.ShapeDtypeStruct((nrows, 128), jnp.float32),
  )(parts128, b2t)


def kernel(x, adj, W1, b1, W2, b2):
  adjv = adj.reshape(2, NBLK, CHUNK).transpose(1, 0, 2)

  zero64 = jnp.zeros((64, 64), jnp.float32)
  wa = jnp.block([[W2[:64], zero64], [zero64, W2[:64]]])
  wb = jnp.block([[W2[64:], zero64], [zero64, W2[64:]]])
  ba = jnp.concatenate([b1[:64], b1[:64]]).reshape(1, 128)
  bb = jnp.concatenate([b1[64:], b1[64:]]).reshape(1, 128)
  b2t = jnp.concatenate([b2, b2]).reshape(1, 128)

  h = _tc_matmul(x, W1)
  h64 = h.reshape(2 * N_NODES, D)
  parts1 = _sc_gather_scatter(h64, adjv, 2, True)
  p1v = parts1.reshape(2, NC, N_PAD // 2, 128)
  g128 = _tc_fuse_relu_matmul(p1v, ba, bb, wa, wb)
  g64 = g128.reshape(N_NODES, D)
  parts2 = _sc_gather_scatter(g64, adjv, 1, False)
  p2v = parts2.reshape(1, NC, N_PAD // 2, 128)
  out128 = _tc_sum_bias(p2v, b2t)
  return out128.reshape(N_NODES, 64)
